# Initial kernel scaffold; baseline (speedup 1.0000x reference)
import jax
import jax.numpy as jnp
from jax.experimental import pallas as pl


def kernel(x, neighborhood_matrix, w1, w2, w3, W_last, b_last):
    raise NotImplementedError("write your pallas kernel here")


# speedup vs baseline: 4.6644x; 4.6644x over previous
import functools

import jax
import jax.numpy as jnp
from jax import lax
from jax.experimental import pallas as pl
from jax.experimental.pallas import tpu as pltpu
from jax.experimental.pallas import tpu_sc as plsc

_B, _N, _D, _K = 8, 1024, 3, 20
_C1, _C2, _C3 = 32, 128, 256
_NCLS = 40
_NK = _N * _K
_ROWS = _B * _NK
_NW = 32
_RPW = _ROWS // _NW
_CHUNK = 128
_NCH = _RPW // _CHUNK


def _sc_gather(table, idx, c):
    mesh = plsc.VectorSubcoreMesh(core_axis_name="c", subcore_axis_name="s")

    @functools.partial(
        pl.kernel,
        mesh=mesh,
        out_type=jax.ShapeDtypeStruct((_ROWS, c), jnp.float32),
        scratch_types=[
            pltpu.VMEM((_NCH, _CHUNK), jnp.int32),
            pltpu.VMEM((2, _CHUNK, c), jnp.float32),
            pltpu.SemaphoreType.DMA,
        ],
    )
    def gather_kernel(table_hbm, idx_hbm, out_hbm, idx_v, rows_v, sem):
        wid = lax.axis_index("s") * 2 + lax.axis_index("c")
        pltpu.sync_copy(idx_hbm.at[wid], idx_v)
        base = wid * _RPW
        pltpu.make_async_copy(
            table_hbm.at[idx_v.at[0]], rows_v.at[0], sem).start()

        def body(j, carry):
            @pl.when(j + 1 < _NCH)
            def _():
                pltpu.make_async_copy(
                    table_hbm.at[idx_v.at[j + 1]], rows_v.at[(j + 1) % 2],
                    sem).start()
            pltpu.make_async_copy(
                table_hbm.at[idx_v.at[j]], rows_v.at[j % 2], sem).wait()
            pltpu.sync_copy(rows_v.at[j % 2],
                            out_hbm.at[pl.ds(base + j * _CHUNK, _CHUNK)])
            return carry

        lax.fori_loop(0, _NCH, body, 0)

    return gather_kernel(table, idx)


def _softmax0(w2d):
    def body(w_ref, o_ref):
        w = w_ref[...]
        m = jnp.max(w, axis=0, keepdims=True)
        e = jnp.exp(w - m)
        o_ref[...] = e / jnp.sum(e, axis=0, keepdims=True)

    return pl.pallas_call(
        body, out_shape=jax.ShapeDtypeStruct(w2d.shape, jnp.float32))(w2d)


def _wfm_matmul(g, w, cin, cout, cpad, nblk):
    def body(g_ref, w_ref, o_ref):
        accs = []
        for d in range(_D):
            acc = None
            for k in range(_K):
                gk = g_ref[0, :, k, d * cin:(d + 1) * cin]
                if cin == 1:
                    t = gk * w_ref[k, 0:1, :]
                else:
                    t = jnp.dot(gk, w_ref[k],
                                preferred_element_type=jnp.float32)
                acc = t if acc is None else acc + t
            accs.append(acc)
        inv = 1.0 / (jnp.sqrt(accs[0] * accs[0] + accs[1] * accs[1]
                              + accs[2] * accs[2]) + 1e-8)
        for d in range(_D):
            o_ref[0, :, d * cout:(d + 1) * cout] = accs[d] * inv

    return pl.pallas_call(
        body,
        grid=(_B, _N // nblk),
        in_specs=[
            pl.BlockSpec((1, nblk, _K, cpad), lambda b, n: (b, n, 0, 0)),
            pl.BlockSpec((_K, cin, cout), lambda b, n: (0, 0, 0)),
        ],
        out_specs=pl.BlockSpec((1, nblk, _D * cout), lambda b, n: (b, n, 0)),
        out_shape=jax.ShapeDtypeStruct((_B, _N, _D * cout), jnp.float32),
    )(g, w)


def _head(h, w_last, b_last):
    def body(h_ref, w_ref, b_ref, o_ref):
        hs = [h_ref[0, :, d * _C3:(d + 1) * _C3] for d in range(_D)]
        ms = [jnp.mean(hd, axis=0, keepdims=True) for hd in hs]
        inv = 1.0 / (jnp.sqrt(ms[0] * ms[0] + ms[1] * ms[1]
                              + ms[2] * ms[2]) + 1e-8)
        ms = [m * inv for m in ms]
        cos = hs[0] * ms[0] + hs[1] * ms[1] + hs[2] * ms[2]
        cos = jnp.clip(cos, -1.0 + 1e-6, 1.0 - 1e-6)
        dist = lax.atan2(jnp.sqrt(1.0 - cos * cos), cos)
        feat = jnp.mean(dist, axis=0, keepdims=True)
        o_ref[0] = jnp.dot(feat, w_ref[...],
                           preferred_element_type=jnp.float32) + b_ref[...]

    out = pl.pallas_call(
        body,
        grid=(_B,),
        in_specs=[
            pl.BlockSpec((1, _N, _D * _C3), lambda b: (b, 0, 0)),
            pl.BlockSpec((_C3, _NCLS), lambda b: (0, 0)),
            pl.BlockSpec((1, _NCLS), lambda b: (0, 0)),
        ],
        out_specs=pl.BlockSpec((1, 1, _NCLS), lambda b: (b, 0, 0)),
        out_shape=jax.ShapeDtypeStruct((_B, 1, _NCLS), jnp.float32),
    )(h, w_last, b_last.reshape(1, _NCLS))
    return out.reshape(_B, _NCLS)


def kernel(x, neighborhood_matrix, w1, w2, w3, W_last, b_last):
    idx = neighborhood_matrix.astype(jnp.int32)
    flat_idx = (idx + (jnp.arange(_B) * _N)[:, None, None]).reshape(
        _NW, _NCH, _CHUNK)

    ws1 = _softmax0(w1.reshape(_K, _C1)).reshape(_K, 1, _C1)
    ws2 = _softmax0(w2.reshape(_K * _C1, _C2)).reshape(_K, _C1, _C2)
    ws3 = _softmax0(w3.reshape(_K * _C2, _C3)).reshape(_K, _C2, _C3)

    table1 = jnp.pad(x.reshape(_B * _N, _D), ((0, 0), (0, 128 - _D)))
    g1 = _sc_gather(table1, flat_idx, 128).reshape(_B, _N, _K, 128)
    h1 = _wfm_matmul(g1, ws1, 1, _C1, 128, 256)

    table2 = jnp.pad(h1.reshape(_B * _N, _D * _C1),
                     ((0, 0), (0, 128 - _D * _C1)))
    g2 = _sc_gather(table2, flat_idx, 128).reshape(_B, _N, _K, 128)
    h2 = _wfm_matmul(g2, ws2, _C1, _C2, 128, 256)

    g3 = _sc_gather(h2.reshape(_B * _N, _D * _C2), flat_idx, _D * _C2)
    g3 = g3.reshape(_B, _N, _K, _D * _C2)
    h3 = _wfm_matmul(g3, ws3, _C2, _C3, _D * _C2, 256)

    return _head(h3, W_last, b_last)


# speedup vs baseline: 12.8382x; 2.7524x over previous
import functools

import jax
import jax.numpy as jnp
from jax import lax
from jax.experimental import pallas as pl
from jax.experimental.pallas import tpu as pltpu
from jax.experimental.pallas import tpu_sc as plsc

_B, _N, _D, _K = 8, 1024, 3, 20
_C1, _C2, _C3 = 32, 128, 256
_NCLS = 40
_NK = _N * _K
_ROWS = _B * _NK
_NW = 32
_RPW = _ROWS // _NW
_CHUNK = 128
_NCH = _RPW // _CHUNK


def _sc_gather(table, idx, c):
    mesh = plsc.VectorSubcoreMesh(core_axis_name="c", subcore_axis_name="s")

    @functools.partial(
        pl.kernel,
        mesh=mesh,
        out_type=jax.ShapeDtypeStruct((_ROWS, c), jnp.float32),
        scratch_types=[
            pltpu.VMEM((_NCH, _CHUNK), jnp.int32),
            pltpu.VMEM((2, _CHUNK, c), jnp.float32),
            pltpu.SemaphoreType.DMA,
        ],
    )
    def gather_kernel(table_hbm, idx_hbm, out_hbm, idx_v, rows_v, sem):
        wid = lax.axis_index("s") * 2 + lax.axis_index("c")
        pltpu.sync_copy(idx_hbm.at[wid], idx_v)
        base = wid * _RPW
        pltpu.make_async_copy(
            table_hbm.at[idx_v.at[0]], rows_v.at[0], sem).start()

        def body(j, carry):
            @pl.when(j + 1 < _NCH)
            def _():
                pltpu.make_async_copy(
                    table_hbm.at[idx_v.at[j + 1]], rows_v.at[(j + 1) % 2],
                    sem).start()
            pltpu.make_async_copy(
                table_hbm.at[idx_v.at[j]], rows_v.at[j % 2], sem).wait()
            pltpu.sync_copy(rows_v.at[j % 2],
                            out_hbm.at[pl.ds(base + j * _CHUNK, _CHUNK)])
            return carry

        lax.fori_loop(0, _NCH, body, 0)

    return gather_kernel(table, idx)


def _softmax0(w2d):
    def body(w_ref, o_ref):
        w = w_ref[...]
        m = jnp.max(w, axis=0, keepdims=True)
        e = jnp.exp(w - m)
        o_ref[...] = e / jnp.sum(e, axis=0, keepdims=True)

    return pl.pallas_call(
        body, out_shape=jax.ShapeDtypeStruct(w2d.shape, jnp.float32))(w2d)


def _wfm_matmul(g, w, cin, cout, cpad, nblk):
    def body(g_ref, w_ref, o_ref):
        accs = []
        for d in range(_D):
            acc = None
            for k in range(_K):
                gk = g_ref[k, 0, :, d * cin:(d + 1) * cin]
                if cin == 1:
                    t = gk * w_ref[k, 0:1, :]
                else:
                    t = jnp.dot(gk, w_ref[k],
                                preferred_element_type=jnp.float32)
                acc = t if acc is None else acc + t
            accs.append(acc)
        inv = 1.0 / (jnp.sqrt(accs[0] * accs[0] + accs[1] * accs[1]
                              + accs[2] * accs[2]) + 1e-8)
        for d in range(_D):
            o_ref[0, :, d * cout:(d + 1) * cout] = accs[d] * inv

    return pl.pallas_call(
        body,
        grid=(_B, _N // nblk),
        in_specs=[
            pl.BlockSpec((_K, 1, nblk, cpad), lambda b, n: (0, b, n, 0)),
            pl.BlockSpec((_K, cin, cout), lambda b, n: (0, 0, 0)),
        ],
        out_specs=pl.BlockSpec((1, nblk, _D * cout), lambda b, n: (b, n, 0)),
        out_shape=jax.ShapeDtypeStruct((_B, _N, _D * cout), jnp.float32),
    )(g, w)


def _head(h, w_last, b_last):
    def body(h_ref, w_ref, b_ref, o_ref):
        hs = [h_ref[0, :, d * _C3:(d + 1) * _C3] for d in range(_D)]
        ms = [jnp.mean(hd, axis=0, keepdims=True) for hd in hs]
        inv = 1.0 / (jnp.sqrt(ms[0] * ms[0] + ms[1] * ms[1]
                              + ms[2] * ms[2]) + 1e-8)
        ms = [m * inv for m in ms]
        cos = hs[0] * ms[0] + hs[1] * ms[1] + hs[2] * ms[2]
        cos = jnp.clip(cos, -1.0 + 1e-6, 1.0 - 1e-6)
        dist = lax.atan2(jnp.sqrt(1.0 - cos * cos), cos)
        feat = jnp.mean(dist, axis=0, keepdims=True)
        o_ref[0] = jnp.dot(feat, w_ref[...],
                           preferred_element_type=jnp.float32) + b_ref[...]

    out = pl.pallas_call(
        body,
        grid=(_B,),
        in_specs=[
            pl.BlockSpec((1, _N, _D * _C3), lambda b: (b, 0, 0)),
            pl.BlockSpec((_C3, _NCLS), lambda b: (0, 0)),
            pl.BlockSpec((1, _NCLS), lambda b: (0, 0)),
        ],
        out_specs=pl.BlockSpec((1, 1, _NCLS), lambda b: (b, 0, 0)),
        out_shape=jax.ShapeDtypeStruct((_B, 1, _NCLS), jnp.float32),
    )(h, w_last, b_last.reshape(1, _NCLS))
    return out.reshape(_B, _NCLS)


def kernel(x, neighborhood_matrix, w1, w2, w3, W_last, b_last):
    idx = neighborhood_matrix.astype(jnp.int32)
    flat_idx = (idx.transpose(2, 0, 1)
                + (jnp.arange(_B) * _N)[None, :, None]).reshape(
        _NW, _NCH, _CHUNK)

    ws1 = _softmax0(w1.reshape(_K, _C1)).reshape(_K, 1, _C1)
    ws2 = _softmax0(w2.reshape(_K * _C1, _C2)).reshape(_K, _C1, _C2)
    ws3 = _softmax0(w3.reshape(_K * _C2, _C3)).reshape(_K, _C2, _C3)

    table1 = jnp.pad(x.reshape(_B * _N, _D), ((0, 0), (0, 128 - _D)))
    g1 = _sc_gather(table1, flat_idx, 128).reshape(_K, _B, _N, 128)
    h1 = _wfm_matmul(g1, ws1, 1, _C1, 128, 256)

    table2 = jnp.pad(h1.reshape(_B * _N, _D * _C1),
                     ((0, 0), (0, 128 - _D * _C1)))
    g2 = _sc_gather(table2, flat_idx, 128).reshape(_K, _B, _N, 128)
    h2 = _wfm_matmul(g2, ws2, _C1, _C2, 128, 256)

    g3 = _sc_gather(h2.reshape(_B * _N, _D * _C2), flat_idx, _D * _C2)
    g3 = g3.reshape(_K, _B, _N, _D * _C2)
    h3 = _wfm_matmul(g3, ws3, _C2, _C3, _D * _C2, 256)

    return _head(h3, W_last, b_last)


# speedup vs baseline: 13.6030x; 1.0596x over previous
import functools

import jax
import jax.numpy as jnp
from jax import lax
from jax.experimental import pallas as pl
from jax.experimental.pallas import tpu as pltpu
from jax.experimental.pallas import tpu_sc as plsc

_B, _N, _D, _K = 8, 1024, 3, 20
_C1, _C2, _C3 = 32, 128, 256
_NCLS = 40
_NK = _N * _K
_ROWS = _B * _NK
_NW = 32
_RPW = _ROWS // _NW
_CHUNK = 128
_NCH = _RPW // _CHUNK


def _sc_gather(table, idx, c):
    mesh = plsc.VectorSubcoreMesh(core_axis_name="c", subcore_axis_name="s")

    @functools.partial(
        pl.kernel,
        mesh=mesh,
        out_type=jax.ShapeDtypeStruct((_ROWS, c), jnp.float32),
        scratch_types=[
            pltpu.VMEM((_NCH, _CHUNK), jnp.int32),
            pltpu.VMEM((2, _CHUNK, c), jnp.float32),
            pltpu.SemaphoreType.DMA,
        ],
    )
    def gather_kernel(table_hbm, idx_hbm, out_hbm, idx_v, rows_v, sem):
        wid = lax.axis_index("s") * 2 + lax.axis_index("c")
        pltpu.sync_copy(idx_hbm.at[wid], idx_v)
        base = wid * _RPW
        pltpu.make_async_copy(
            table_hbm.at[idx_v.at[0]], rows_v.at[0], sem).start()

        def body(j, carry):
            @pl.when(j + 1 < _NCH)
            def _():
                pltpu.make_async_copy(
                    table_hbm.at[idx_v.at[j + 1]], rows_v.at[(j + 1) % 2],
                    sem).start()
            pltpu.make_async_copy(
                table_hbm.at[idx_v.at[j]], rows_v.at[j % 2], sem).wait()
            pltpu.sync_copy(rows_v.at[j % 2],
                            out_hbm.at[pl.ds(base + j * _CHUNK, _CHUNK)])
            return carry

        lax.fori_loop(0, _NCH, body, 0)

    return gather_kernel(table, idx)


def _softmax0(w2d):
    def body(w_ref, o_ref):
        w = w_ref[...]
        m = jnp.max(w, axis=0, keepdims=True)
        e = jnp.exp(w - m)
        o_ref[...] = e / jnp.sum(e, axis=0, keepdims=True)

    return pl.pallas_call(
        body, out_shape=jax.ShapeDtypeStruct(w2d.shape, jnp.float32))(w2d)


def _wfm_folded(g, wbig, cout, nblk):
    cpad = g.shape[-1]

    def body(g_ref, w_ref, o_ref):
        acc = jnp.dot(g_ref[0, 0], w_ref[0],
                      preferred_element_type=jnp.float32)
        for k in range(1, _K):
            acc = acc + jnp.dot(g_ref[k, 0], w_ref[k],
                                preferred_element_type=jnp.float32)
        s = [acc[:, d * cout:(d + 1) * cout] for d in range(_D)]
        inv = 1.0 / (jnp.sqrt(s[0] * s[0] + s[1] * s[1] + s[2] * s[2])
                     + 1e-8)
        for d in range(_D):
            o_ref[0, :, d * cout:(d + 1) * cout] = s[d] * inv

    return pl.pallas_call(
        body,
        grid=(_B, _N // nblk),
        in_specs=[
            pl.BlockSpec((_K, 1, nblk, cpad), lambda b, n: (0, b, n, 0)),
            pl.BlockSpec((_K, cpad, _D * cout), lambda b, n: (0, 0, 0)),
        ],
        out_specs=pl.BlockSpec((1, nblk, _D * cout), lambda b, n: (b, n, 0)),
        out_shape=jax.ShapeDtypeStruct((_B, _N, _D * cout), jnp.float32),
    )(g, wbig)


def _wfm_perd(g, w, cin, cout, nblk):
    def body(g_ref, w_ref, o_ref):
        accs = []
        for d in range(_D):
            acc = None
            for k in range(_K):
                gk = g_ref[k, 0, :, d * cin:(d + 1) * cin]
                t = jnp.dot(gk, w_ref[k], preferred_element_type=jnp.float32)
                acc = t if acc is None else acc + t
            accs.append(acc)
        inv = 1.0 / (jnp.sqrt(accs[0] * accs[0] + accs[1] * accs[1]
                              + accs[2] * accs[2]) + 1e-8)
        for d in range(_D):
            o_ref[0, :, d * cout:(d + 1) * cout] = accs[d] * inv

    return pl.pallas_call(
        body,
        grid=(_B, _N // nblk),
        in_specs=[
            pl.BlockSpec((_K, 1, nblk, _D * cin), lambda b, n: (0, b, n, 0)),
            pl.BlockSpec((_K, cin, cout), lambda b, n: (0, 0, 0)),
        ],
        out_specs=pl.BlockSpec((1, nblk, _D * cout), lambda b, n: (b, n, 0)),
        out_shape=jax.ShapeDtypeStruct((_B, _N, _D * cout), jnp.float32),
    )(g, w)


def _head(h, w_last, b_last):
    def body(h_ref, w_ref, b_ref, o_ref):
        hs = [h_ref[0, :, d * _C3:(d + 1) * _C3] for d in range(_D)]
        ms = [jnp.mean(hd, axis=0, keepdims=True) for hd in hs]
        inv = 1.0 / (jnp.sqrt(ms[0] * ms[0] + ms[1] * ms[1]
                              + ms[2] * ms[2]) + 1e-8)
        ms = [m * inv for m in ms]
        cos = hs[0] * ms[0] + hs[1] * ms[1] + hs[2] * ms[2]
        cos = jnp.clip(cos, -1.0 + 1e-6, 1.0 - 1e-6)
        dist = lax.atan2(jnp.sqrt(1.0 - cos * cos), cos)
        feat = jnp.mean(dist, axis=0, keepdims=True)
        o_ref[0] = jnp.dot(feat, w_ref[...],
                           preferred_element_type=jnp.float32) + b_ref[...]

    out = pl.pallas_call(
        body,
        grid=(_B,),
        in_specs=[
            pl.BlockSpec((1, _N, _D * _C3), lambda b: (b, 0, 0)),
            pl.BlockSpec((_C3, _NCLS), lambda b: (0, 0)),
            pl.BlockSpec((1, _NCLS), lambda b: (0, 0)),
        ],
        out_specs=pl.BlockSpec((1, 1, _NCLS), lambda b: (b, 0, 0)),
        out_shape=jax.ShapeDtypeStruct((_B, 1, _NCLS), jnp.float32),
    )(h, w_last, b_last.reshape(1, _NCLS))
    return out.reshape(_B, _NCLS)


def kernel(x, neighborhood_matrix, w1, w2, w3, W_last, b_last):
    idx = neighborhood_matrix.astype(jnp.int32)
    flat_idx = (idx.transpose(2, 0, 1)
                + (jnp.arange(_B) * _N)[None, :, None]).reshape(
        _NW, _NCH, _CHUNK)

    ws1 = _softmax0(w1.reshape(_K, _C1))
    ws2 = _softmax0(w2.reshape(_K * _C1, _C2)).reshape(_K, _C1, _C2)
    ws3 = _softmax0(w3.reshape(_K * _C2, _C3)).reshape(_K, _C2, _C3)

    w1big = jnp.zeros((_K, 128, _D * _C1), jnp.float32)
    w2big = jnp.zeros((_K, 128, _D * _C2), jnp.float32)
    for d in range(_D):
        w1big = w1big.at[:, d, d * _C1:(d + 1) * _C1].set(ws1)
        w2big = w2big.at[:, d * _C1:(d + 1) * _C1,
                         d * _C2:(d + 1) * _C2].set(ws2)

    table1 = jnp.pad(x.reshape(_B * _N, _D), ((0, 0), (0, 128 - _D)))
    g1 = _sc_gather(table1, flat_idx, 128).reshape(_K, _B, _N, 128)
    h1 = _wfm_folded(g1, w1big, _C1, 256)

    table2 = jnp.pad(h1.reshape(_B * _N, _D * _C1),
                     ((0, 0), (0, 128 - _D * _C1)))
    g2 = _sc_gather(table2, flat_idx, 128).reshape(_K, _B, _N, 128)
    h2 = _wfm_folded(g2, w2big, _C2, 256)

    g3 = _sc_gather(h2.reshape(_B * _N, _D * _C2), flat_idx, _D * _C2)
    g3 = g3.reshape(_K, _B, _N, _D * _C2)
    h3 = _wfm_perd(g3, ws3, _C2, _C3, 256)

    return _head(h3, W_last, b_last)
